# Initial kernel scaffold; baseline (speedup 1.0000x reference)
import jax
import jax.numpy as jnp
from jax.experimental import pallas as pl


def kernel(x, node_ts, edge_index_a, edge_index_b, W1, b1, W2, b2, fc1_W, fc1_b, fc2_W, fc2_b):
    raise NotImplementedError("write your pallas kernel here")


# speedup vs baseline: 14.2158x; 14.2158x over previous
import functools

import jax
import jax.numpy as jnp
from jax import lax
from jax.experimental import pallas as pl
from jax.experimental.pallas import tpu as pltpu
from jax.experimental.pallas import tpu_sc as plsc

NC, NS, LANES = 2, 16, 16
NW = NC * NS
CHUNK = 128
N_NODES = 10000
DUMP = N_NODES
ACC_R = 10240
TS_PAD = 10016
BLK = 1024
E_TOT = 640000
E_PAD = 643072
EPW = E_PAD // NW
NCHUNK = EPW // CHUNK
ZROWS = ACC_R // NS

_vmesh = plsc.VectorSubcoreMesh(core_axis_name="c", subcore_axis_name="s")


def _wid():
    return lax.axis_index("s") * NC + lax.axis_index("c")


def _mask_body(ts_hbm, src_hbm, dst_hbm, eff_hbm, ts_v, src_v, dst_v, eff_v):
    base = _wid() * EPW
    pltpu.sync_copy(ts_hbm, ts_v)
    pltpu.sync_copy(src_hbm.at[pl.ds(base, EPW)], src_v)
    pltpu.sync_copy(dst_hbm.at[pl.ds(base, EPW)], dst_v)
    dump_v = jnp.full((LANES,), DUMP, jnp.int32)

    def body(i, carry):
        off = i * LANES
        sv = src_v[pl.ds(off, LANES)]
        dv = dst_v[pl.ds(off, LANES)]
        ts_s = plsc.load_gather(ts_v, [sv])
        ts_d = plsc.load_gather(ts_v, [dv])
        eff_v[pl.ds(off, LANES)] = jnp.where(ts_s <= ts_d, dv, dump_v)
        return carry

    lax.fori_loop(0, EPW // LANES, body, 0)
    pltpu.sync_copy(eff_v, eff_hbm.at[pl.ds(base, EPW)])


_mask_kernel = functools.partial(
    pl.kernel,
    out_type=jax.ShapeDtypeStruct((E_PAD,), jnp.int32),
    mesh=_vmesh,
    compiler_params=pltpu.CompilerParams(needs_layout_passes=False),
    scratch_types=[
        pltpu.VMEM((TS_PAD,), jnp.float32),
        pltpu.VMEM((EPW,), jnp.int32),
        pltpu.VMEM((EPW,), jnp.int32),
        pltpu.VMEM((EPW,), jnp.int32),
    ],
)(_mask_body)


def _scatter_body(feat_hbm, src_hbm, eff_hbm, zeros_hbm, out_hbm,
                  acc_sh, rows_v, sidx_v, didx_v, sem):
    c = lax.axis_index("c")
    t = lax.axis_index("s")
    wid = t * NC + c
    pltpu.sync_copy(zeros_hbm, acc_sh.at[pl.ds(t * ZROWS, ZROWS)])
    plsc.subcore_barrier()

    base0 = wid * EPW

    def body(i, carry):
        b = base0 + i * CHUNK
        pltpu.sync_copy(src_hbm.at[pl.ds(b, CHUNK)], sidx_v)
        pltpu.sync_copy(eff_hbm.at[pl.ds(b, CHUNK)], didx_v)
        pltpu.async_copy(feat_hbm.at[sidx_v], rows_v, sem).wait()
        pltpu.sync_copy(rows_v, acc_sh.at[didx_v], add=True)
        return carry

    lax.fori_loop(0, NCHUNK, body, 0)
    plsc.subcore_barrier()
    pltpu.sync_copy(acc_sh.at[pl.ds(t * ZROWS, ZROWS)],
                    out_hbm.at[pl.ds(c * ACC_R + t * ZROWS, ZROWS)])


def _make_scatter(feat_rows):
    return functools.partial(
        pl.kernel,
        out_type=jax.ShapeDtypeStruct((NC * ACC_R, 128), jnp.float32),
        mesh=_vmesh,
        scratch_types=[
            pltpu.VMEM_SHARED((ACC_R, 128), jnp.float32),
            pltpu.VMEM((CHUNK, 128), jnp.float32),
            pltpu.VMEM((CHUNK,), jnp.int32),
            pltpu.VMEM((CHUNK,), jnp.int32),
            pltpu.SemaphoreType.DMA,
        ],
    )(_scatter_body)


_scatter_kernel = _make_scatter(N_NODES)


def _layer_body(p0, p1, w, b, o):
    s = (p0[...] + p1[...]) * 0.5
    o[...] = jnp.maximum(
        jnp.dot(s, w[...], preferred_element_type=jnp.float32) + b[...], 0.0)


def _layer_tc(parts, w, b):
    grid = ACC_R // BLK
    return pl.pallas_call(
        _layer_body,
        grid=(grid,),
        in_specs=[
            pl.BlockSpec((BLK, 128), lambda i: (i, 0)),
            pl.BlockSpec((BLK, 128), lambda i: (i + grid, 0)),
            pl.BlockSpec((128, 128), lambda i: (0, 0)),
            pl.BlockSpec((1, 128), lambda i: (0, 0)),
        ],
        out_specs=pl.BlockSpec((BLK, 128), lambda i: (i, 0)),
        out_shape=jax.ShapeDtypeStruct((ACC_R, 128), jnp.float32),
    )(parts, parts, w, b.reshape(1, 128))


def _head_body(p0, p1, w2, b2, f1w, f1b, f2w, f2b, o):
    s = (p0[...] + p1[...]) * 0.5
    h = jnp.maximum(
        jnp.dot(s, w2[...], preferred_element_type=jnp.float32) + b2[...], 0.0)
    f = jnp.maximum(
        jnp.dot(h, f1w[...], preferred_element_type=jnp.float32) + f1b[...], 0.0)
    o[...] = jnp.sum(f * f2w[...], axis=1, keepdims=True) + f2b[...]


def _head_tc(parts, w2, b2, f1w, f1b, f2w, f2b):
    grid = ACC_R // BLK
    return pl.pallas_call(
        _head_body,
        grid=(grid,),
        in_specs=[
            pl.BlockSpec((BLK, 128), lambda i: (i, 0)),
            pl.BlockSpec((BLK, 128), lambda i: (i + grid, 0)),
            pl.BlockSpec((128, 128), lambda i: (0, 0)),
            pl.BlockSpec((1, 128), lambda i: (0, 0)),
            pl.BlockSpec((128, 64), lambda i: (0, 0)),
            pl.BlockSpec((1, 64), lambda i: (0, 0)),
            pl.BlockSpec((1, 64), lambda i: (0, 0)),
            pl.BlockSpec((1, 1), lambda i: (0, 0)),
        ],
        out_specs=pl.BlockSpec((BLK, 1), lambda i: (i, 0)),
        out_shape=jax.ShapeDtypeStruct((ACC_R, 1), jnp.float32),
    )(parts, parts, w2, b2.reshape(1, 128), f1w, f1b.reshape(1, 64),
      f2w.reshape(1, 64), f2b.reshape(1, 1))


def kernel(x, node_ts, edge_index_a, edge_index_b,
           W1, b1, W2, b2, fc1_W, fc1_b, fc2_W, fc2_b):
    pad_e = E_PAD - E_TOT
    src = jnp.concatenate([edge_index_a[0], edge_index_b[0],
                           jnp.zeros((pad_e,), jnp.int32)])
    dst = jnp.concatenate([edge_index_a[1], edge_index_b[1],
                           jnp.full((pad_e,), DUMP, jnp.int32)])
    ts_p = jnp.pad(node_ts, (0, TS_PAD - N_NODES))
    zeros = jnp.zeros((ZROWS, 128), jnp.float32)

    eff = _mask_kernel(ts_p, src, dst)

    parts1 = _scatter_kernel(x, src, eff, zeros)
    h1 = _layer_tc(parts1, W1, b1)
    parts2 = _scatter_kernel(h1, src, eff, zeros)
    out = _head_tc(parts2, W2, b2, fc1_W, fc1_b, fc2_W, fc2_b)
    return out[:N_NODES]


# speedup vs baseline: 15.9596x; 1.1227x over previous
import functools

import jax
import jax.numpy as jnp
from jax import lax
from jax.experimental import pallas as pl
from jax.experimental.pallas import tpu as pltpu
from jax.experimental.pallas import tpu_sc as plsc

NC, NS, LANES = 2, 16, 16
NW = NC * NS
CHUNK = 64
N_NODES = 10000
DUMP = N_NODES
ACC_R = 10240
TS_PAD = 10016
BLK = 1024
E_TOT = 640000
EPW = 20480
E_PAD = NW * EPW
EPW_C = EPW + 8 * CHUNK
ZROWS = ACC_R // NS

_vmesh = plsc.VectorSubcoreMesh(core_axis_name="c", subcore_axis_name="s")


def _wid():
    return lax.axis_index("s") * NC + lax.axis_index("c")


def _mask_body(ts_hbm, src_hbm, dst_hbm, srcc_hbm, effc_hbm, cnt_hbm,
               ts_v, src_v, dst_v, srcc_v, effc_v, cnt_v):
    wid = _wid()
    base = wid * EPW
    pltpu.sync_copy(ts_hbm, ts_v)
    pltpu.sync_copy(src_hbm.at[pl.ds(base, EPW)], src_v)
    pltpu.sync_copy(dst_hbm.at[pl.ds(base, EPW)], dst_v)
    zero16 = jnp.zeros((LANES,), jnp.int32)
    dump16 = jnp.full((LANES,), DUMP, jnp.int32)

    def memset_body(i, c):
        off = i * LANES
        srcc_v[pl.ds(off, LANES)] = zero16
        effc_v[pl.ds(off, LANES)] = dump16
        return c

    lax.fori_loop(0, EPW_C // LANES, memset_body, 0)

    def body(i, off):
        o = i * LANES
        sv = src_v[pl.ds(o, LANES)]
        dv = dst_v[pl.ds(o, LANES)]
        ts_s = plsc.load_gather(ts_v, [sv])
        ts_d = plsc.load_gather(ts_v, [dv])
        keep = ts_s <= ts_d
        plsc.store_compressed(srcc_v.at[pl.ds(off, LANES)], sv, mask=keep)
        plsc.store_compressed(effc_v.at[pl.ds(off, LANES)], dv, mask=keep)
        return off + jnp.max(plsc.all_reduce_population_count(keep))

    cnt = lax.fori_loop(0, EPW // LANES, body, jnp.int32(0))
    cnt_v[...] = jnp.full((LANES,), cnt, jnp.int32)
    cbase = wid * EPW_C
    pltpu.sync_copy(srcc_v, srcc_hbm.at[pl.ds(cbase, EPW_C)])
    pltpu.sync_copy(effc_v, effc_hbm.at[pl.ds(cbase, EPW_C)])
    pltpu.sync_copy(cnt_v, cnt_hbm.at[pl.ds(wid * LANES, LANES)])


_mask_kernel = functools.partial(
    pl.kernel,
    out_type=(
        jax.ShapeDtypeStruct((NW * EPW_C,), jnp.int32),
        jax.ShapeDtypeStruct((NW * EPW_C,), jnp.int32),
        jax.ShapeDtypeStruct((NW * LANES,), jnp.int32),
    ),
    mesh=_vmesh,
    compiler_params=pltpu.CompilerParams(needs_layout_passes=False),
    scratch_types=[
        pltpu.VMEM((TS_PAD,), jnp.float32),
        pltpu.VMEM((EPW,), jnp.int32),
        pltpu.VMEM((EPW,), jnp.int32),
        pltpu.VMEM((EPW_C,), jnp.int32),
        pltpu.VMEM((EPW_C,), jnp.int32),
        pltpu.VMEM((LANES,), jnp.int32),
    ],
)(_mask_body)


def _scatter_body(feat_hbm, srcc_hbm, effc_hbm, cnt_hbm, zeros_hbm, out_hbm,
                  acc_sh, rows_v, sbuf_v, dbuf_v, cnt_v,
                  si0, si1, si2, si3, sg0, sg1, sg2, sg3):
    c = lax.axis_index("c")
    t = lax.axis_index("s")
    wid = t * NC + c
    si = (si0, si1, si2, si3)
    sg = (sg0, sg1, sg2, sg3)
    pltpu.sync_copy(zeros_hbm, acc_sh.at[pl.ds(t * ZROWS, ZROWS)])
    pltpu.sync_copy(cnt_hbm.at[pl.ds(wid * LANES, LANES)], cnt_v)
    cnt = jnp.max(cnt_v[...])
    k4 = (cnt + (4 * CHUNK - 1)) // (4 * CHUNK)
    plsc.subcore_barrier()

    cbase = wid * EPW_C

    def issue_idx(chunk, u):
        b = cbase + chunk * CHUNK
        pltpu.async_copy(srcc_hbm.at[pl.ds(b, CHUNK)], sbuf_v.at[u], si[u])
        pltpu.async_copy(effc_hbm.at[pl.ds(b, CHUNK)], dbuf_v.at[u], si[u])

    def wait_idx(u):
        pltpu.make_async_copy(srcc_hbm.at[pl.ds(0, CHUNK)], sbuf_v.at[u], si[u]).wait()
        pltpu.make_async_copy(effc_hbm.at[pl.ds(0, CHUNK)], dbuf_v.at[u], si[u]).wait()

    def issue_gather(u):
        pltpu.async_copy(feat_hbm.at[sbuf_v.at[u]], rows_v.at[u], sg[u])

    def wait_gather(u):
        pltpu.make_async_copy(feat_hbm.at[sbuf_v.at[u]], rows_v.at[u], sg[u]).wait()

    for u in range(4):
        issue_idx(u, u)
    for u in range(3):
        wait_idx(u)
        issue_gather(u)

    def body4(k, carry):
        for u in range(4):
            j = k * 4 + u
            b3 = (u + 3) % 4
            wait_gather(u)
            pltpu.sync_copy(rows_v.at[u], acc_sh.at[dbuf_v.at[u]], add=True)
            issue_idx(j + 4, u)
            wait_idx(b3)
            issue_gather(b3)
        return carry

    lax.fori_loop(0, k4, body4, 0)

    for u in range(3):
        wait_gather(u)
    wait_idx(3)
    plsc.subcore_barrier()
    pltpu.sync_copy(acc_sh.at[pl.ds(t * ZROWS, ZROWS)],
                    out_hbm.at[pl.ds(c * ACC_R + t * ZROWS, ZROWS)])


_scatter_kernel = functools.partial(
    pl.kernel,
    out_type=jax.ShapeDtypeStruct((NC * ACC_R, 128), jnp.float32),
    mesh=_vmesh,
    compiler_params=pltpu.CompilerParams(needs_layout_passes=False),
    scratch_types=[
        pltpu.VMEM_SHARED((ACC_R, 128), jnp.float32),
        pltpu.VMEM((4, CHUNK, 128), jnp.float32),
        pltpu.VMEM((4, CHUNK), jnp.int32),
        pltpu.VMEM((4, CHUNK), jnp.int32),
        pltpu.VMEM((LANES,), jnp.int32),
        pltpu.SemaphoreType.DMA,
        pltpu.SemaphoreType.DMA,
        pltpu.SemaphoreType.DMA,
        pltpu.SemaphoreType.DMA,
        pltpu.SemaphoreType.DMA,
        pltpu.SemaphoreType.DMA,
        pltpu.SemaphoreType.DMA,
        pltpu.SemaphoreType.DMA,
    ],
)(_scatter_body)


def _layer_body(p0, p1, w, b, o):
    s = (p0[...] + p1[...]) * 0.5
    o[...] = jnp.maximum(
        jnp.dot(s, w[...], preferred_element_type=jnp.float32) + b[...], 0.0)


def _layer_tc(parts, w, b):
    grid = ACC_R // BLK
    return pl.pallas_call(
        _layer_body,
        grid=(grid,),
        in_specs=[
            pl.BlockSpec((BLK, 128), lambda i: (i, 0)),
            pl.BlockSpec((BLK, 128), lambda i: (i + grid, 0)),
            pl.BlockSpec((128, 128), lambda i: (0, 0)),
            pl.BlockSpec((1, 128), lambda i: (0, 0)),
        ],
        out_specs=pl.BlockSpec((BLK, 128), lambda i: (i, 0)),
        out_shape=jax.ShapeDtypeStruct((ACC_R, 128), jnp.float32),
    )(parts, parts, w, b.reshape(1, 128))


def _head_body(p0, p1, w2, b2, f1w, f1b, f2w, f2b, o):
    s = (p0[...] + p1[...]) * 0.5
    h = jnp.maximum(
        jnp.dot(s, w2[...], preferred_element_type=jnp.float32) + b2[...], 0.0)
    f = jnp.maximum(
        jnp.dot(h, f1w[...], preferred_element_type=jnp.float32) + f1b[...], 0.0)
    o[...] = jnp.sum(f * f2w[...], axis=1, keepdims=True) + f2b[...]


def _head_tc(parts, w2, b2, f1w, f1b, f2w, f2b):
    grid = ACC_R // BLK
    return pl.pallas_call(
        _head_body,
        grid=(grid,),
        in_specs=[
            pl.BlockSpec((BLK, 128), lambda i: (i, 0)),
            pl.BlockSpec((BLK, 128), lambda i: (i + grid, 0)),
            pl.BlockSpec((128, 128), lambda i: (0, 0)),
            pl.BlockSpec((1, 128), lambda i: (0, 0)),
            pl.BlockSpec((128, 64), lambda i: (0, 0)),
            pl.BlockSpec((1, 64), lambda i: (0, 0)),
            pl.BlockSpec((1, 64), lambda i: (0, 0)),
            pl.BlockSpec((1, 1), lambda i: (0, 0)),
        ],
        out_specs=pl.BlockSpec((BLK, 1), lambda i: (i, 0)),
        out_shape=jax.ShapeDtypeStruct((ACC_R, 1), jnp.float32),
    )(parts, parts, w2, b2.reshape(1, 128), f1w, f1b.reshape(1, 64),
      f2w.reshape(1, 64), f2b.reshape(1, 1))


def kernel(x, node_ts, edge_index_a, edge_index_b,
           W1, b1, W2, b2, fc1_W, fc1_b, fc2_W, fc2_b):
    pad_e = E_PAD - E_TOT
    src = jnp.concatenate([edge_index_a[0], edge_index_b[0],
                           jnp.zeros((pad_e,), jnp.int32)])
    dst = jnp.concatenate([edge_index_a[1], edge_index_b[1],
                           jnp.full((pad_e,), DUMP, jnp.int32)])
    ts_p = jnp.pad(node_ts, (0, TS_PAD - N_NODES))
    zeros = jnp.zeros((ZROWS, 128), jnp.float32)

    srcc, effc, cnts = _mask_kernel(ts_p, src, dst)

    parts1 = _scatter_kernel(x, srcc, effc, cnts, zeros)
    h1 = _layer_tc(parts1, W1, b1)
    parts2 = _scatter_kernel(h1, srcc, effc, cnts, zeros)
    out = _head_tc(parts2, W2, b2, fc1_W, fc1_b, fc2_W, fc2_b)
    return out[:N_NODES]


# speedup vs baseline: 42.1697x; 2.6423x over previous
import functools

import jax
import jax.numpy as jnp
from jax import lax
from jax.experimental import pallas as pl
from jax.experimental.pallas import tpu as pltpu
from jax.experimental.pallas import tpu_sc as plsc

NC, NS, LANES = 2, 16, 16
NW = NC * NS
CHUNK = 128
HCH = 64
N_NODES = 10000
DUMP = N_NODES
ACC_R = 10112
TS_PAD = 10016
BLK = 632
E_TOT = 640000
EPW = 20480
E_PAD = NW * EPW
EPW_C = EPW + 5 * CHUNK
ZROWS = ACC_R // NS
TAB_R = ACC_R
TROWS = TAB_R // NS

_vmesh = plsc.VectorSubcoreMesh(core_axis_name="c", subcore_axis_name="s")


def _wid():
    return lax.axis_index("s") * NC + lax.axis_index("c")


def _mask_body(ts_hbm, src_hbm, dst_hbm, srcc_hbm, effc_hbm, cnt_hbm,
               ts_v, src_v, dst_v, srcc_v, effc_v, cnt_v):
    wid = _wid()
    base = wid * EPW
    pltpu.sync_copy(ts_hbm, ts_v)
    pltpu.sync_copy(src_hbm.at[pl.ds(base, EPW)], src_v)
    pltpu.sync_copy(dst_hbm.at[pl.ds(base, EPW)], dst_v)
    zero16 = jnp.zeros((LANES,), jnp.int32)
    dump16 = jnp.full((LANES,), DUMP, jnp.int32)

    def memset_body(i, c):
        off = i * LANES
        srcc_v[pl.ds(off, LANES)] = zero16
        effc_v[pl.ds(off, LANES)] = dump16
        return c

    lax.fori_loop(0, EPW_C // LANES, memset_body, 0)

    def body(i, off):
        o = i * LANES
        sv = src_v[pl.ds(o, LANES)]
        dv = dst_v[pl.ds(o, LANES)]
        ts_s = plsc.load_gather(ts_v, [sv])
        ts_d = plsc.load_gather(ts_v, [dv])
        keep = ts_s <= ts_d
        plsc.store_compressed(srcc_v.at[pl.ds(off, LANES)], sv, mask=keep)
        plsc.store_compressed(effc_v.at[pl.ds(off, LANES)], dv, mask=keep)
        return off + jnp.max(plsc.all_reduce_population_count(keep))

    cnt = lax.fori_loop(0, EPW // LANES, body, jnp.int32(0))
    cnt16 = jnp.full((LANES,), cnt, jnp.int32)
    for q in range(CHUNK // LANES):
        cnt_v[pl.ds(q * LANES, LANES)] = cnt16
    cbase = wid * EPW_C
    pltpu.sync_copy(srcc_v, srcc_hbm.at[pl.ds(cbase, EPW_C)])
    pltpu.sync_copy(effc_v, effc_hbm.at[pl.ds(cbase, EPW_C)])
    pltpu.sync_copy(cnt_v, cnt_hbm.at[pl.ds(wid * CHUNK, CHUNK)])


_mask_kernel = functools.partial(
    pl.kernel,
    out_type=(
        jax.ShapeDtypeStruct((NW * EPW_C,), jnp.int32),
        jax.ShapeDtypeStruct((NW * EPW_C,), jnp.int32),
        jax.ShapeDtypeStruct((NW * CHUNK,), jnp.int32),
    ),
    mesh=_vmesh,
    compiler_params=pltpu.CompilerParams(needs_layout_passes=False),
    scratch_types=[
        pltpu.VMEM((TS_PAD,), jnp.float32),
        pltpu.VMEM((EPW,), jnp.int32),
        pltpu.VMEM((EPW,), jnp.int32),
        pltpu.VMEM((EPW_C,), jnp.int32),
        pltpu.VMEM((EPW_C,), jnp.int32),
        pltpu.VMEM((CHUNK,), jnp.int32),
    ],
)(_mask_body)


def _scatter_body(feat_hbm, srcc_hbm, effc_hbm, cnt_hbm, zeros_hbm, out_hbm,
                  acc_sh, tab_sh, rows_v, sbuf_v, dbuf_v,
                  si0, si1, si2, sg0, sg1, sg2):
    c = lax.axis_index("c")
    t = lax.axis_index("s")
    si = (si0, si1, si2)
    sg = (sg0, sg1, sg2)
    pltpu.sync_copy(feat_hbm.at[pl.ds(t * TROWS, TROWS), pl.ds(c * HCH, HCH)],
                    tab_sh.at[pl.ds(t * TROWS, TROWS)])
    pltpu.sync_copy(zeros_hbm, acc_sh.at[pl.ds(t * ZROWS, ZROWS)])
    plsc.subcore_barrier()

    def issue_idx(cbase, chunk, u):
        b = cbase + chunk * CHUNK
        pltpu.async_copy(srcc_hbm.at[pl.ds(b, CHUNK)], sbuf_v.at[u], si[u])
        pltpu.async_copy(effc_hbm.at[pl.ds(b, CHUNK)], dbuf_v.at[u], si[u])

    def wait_idx(u):
        pltpu.make_async_copy(srcc_hbm.at[pl.ds(0, CHUNK)], sbuf_v.at[u], si[u]).wait()
        pltpu.make_async_copy(effc_hbm.at[pl.ds(0, CHUNK)], dbuf_v.at[u], si[u]).wait()

    def issue_gather(u):
        pltpu.async_copy(tab_sh.at[sbuf_v.at[u]], rows_v.at[u], sg[u])

    def wait_gather(u):
        pltpu.make_async_copy(tab_sh.at[sbuf_v.at[u]], rows_v.at[u], sg[u]).wait()

    for reg in range(2):
        wid = t * NC + reg
        cbase = wid * EPW_C
        pltpu.sync_copy(cnt_hbm.at[pl.ds(wid * CHUNK, CHUNK)], sbuf_v.at[0])
        cnt = jnp.max(sbuf_v[0, pl.ds(0, LANES)])
        k3 = (cnt + (3 * CHUNK - 1)) // (3 * CHUNK)

        for u in range(3):
            issue_idx(cbase, u, u)
        for u in range(2):
            wait_idx(u)
            issue_gather(u)

        def body3(k, carry):
            for u in range(3):
                j = k * 3 + u
                b2 = (u + 2) % 3
                wait_gather(u)
                pltpu.sync_copy(rows_v.at[u], acc_sh.at[dbuf_v.at[u]], add=True)
                issue_idx(cbase, j + 3, u)
                wait_idx(b2)
                issue_gather(b2)
            return carry

        lax.fori_loop(0, k3, body3, 0)

        for u in range(2):
            wait_gather(u)
        wait_idx(2)

    plsc.subcore_barrier()
    pltpu.sync_copy(acc_sh.at[pl.ds(t * ZROWS, ZROWS)],
                    out_hbm.at[pl.ds(t * ZROWS, ZROWS), pl.ds(c * HCH, HCH)])


_scatter_kernel = functools.partial(
    pl.kernel,
    out_type=jax.ShapeDtypeStruct((ACC_R, 128), jnp.float32),
    mesh=_vmesh,
    compiler_params=pltpu.CompilerParams(needs_layout_passes=False,
                                         use_tc_tiling_on_sc=False),
    scratch_types=[
        pltpu.VMEM_SHARED((ACC_R, HCH), jnp.float32),
        pltpu.VMEM_SHARED((TAB_R, HCH), jnp.float32),
        pltpu.VMEM((3, CHUNK, HCH), jnp.float32),
        pltpu.VMEM((3, CHUNK), jnp.int32),
        pltpu.VMEM((3, CHUNK), jnp.int32),
        pltpu.SemaphoreType.DMA,
        pltpu.SemaphoreType.DMA,
        pltpu.SemaphoreType.DMA,
        pltpu.SemaphoreType.DMA,
        pltpu.SemaphoreType.DMA,
        pltpu.SemaphoreType.DMA,
    ],
)(_scatter_body)


def _layer_body(p, w, b, o):
    s = p[...] * 0.5
    o[...] = jnp.maximum(
        jnp.dot(s, w[...], preferred_element_type=jnp.float32) + b[...], 0.0)


def _layer_tc(parts, w, b):
    grid = ACC_R // BLK
    return pl.pallas_call(
        _layer_body,
        grid=(grid,),
        in_specs=[
            pl.BlockSpec((BLK, 128), lambda i: (i, 0)),
            pl.BlockSpec((128, 128), lambda i: (0, 0)),
            pl.BlockSpec((1, 128), lambda i: (0, 0)),
        ],
        out_specs=pl.BlockSpec((BLK, 128), lambda i: (i, 0)),
        out_shape=jax.ShapeDtypeStruct((ACC_R, 128), jnp.float32),
    )(parts, w, b.reshape(1, 128))


def _head_body(p, w2, b2, f1w, f1b, f2w, f2b, o):
    s = p[...] * 0.5
    h = jnp.maximum(
        jnp.dot(s, w2[...], preferred_element_type=jnp.float32) + b2[...], 0.0)
    f = jnp.maximum(
        jnp.dot(h, f1w[...], preferred_element_type=jnp.float32) + f1b[...], 0.0)
    o[...] = jnp.sum(f * f2w[...], axis=1, keepdims=True) + f2b[...]


def _head_tc(parts, w2, b2, f1w, f1b, f2w, f2b):
    grid = ACC_R // BLK
    return pl.pallas_call(
        _head_body,
        grid=(grid,),
        in_specs=[
            pl.BlockSpec((BLK, 128), lambda i: (i, 0)),
            pl.BlockSpec((128, 128), lambda i: (0, 0)),
            pl.BlockSpec((1, 128), lambda i: (0, 0)),
            pl.BlockSpec((128, 64), lambda i: (0, 0)),
            pl.BlockSpec((1, 64), lambda i: (0, 0)),
            pl.BlockSpec((1, 64), lambda i: (0, 0)),
            pl.BlockSpec((1, 1), lambda i: (0, 0)),
        ],
        out_specs=pl.BlockSpec((BLK, 1), lambda i: (i, 0)),
        out_shape=jax.ShapeDtypeStruct((ACC_R, 1), jnp.float32),
    )(parts, w2, b2.reshape(1, 128), f1w, f1b.reshape(1, 64),
      f2w.reshape(1, 64), f2b.reshape(1, 1))


def kernel(x, node_ts, edge_index_a, edge_index_b,
           W1, b1, W2, b2, fc1_W, fc1_b, fc2_W, fc2_b):
    pad_e = E_PAD - E_TOT
    src = jnp.concatenate([edge_index_a[0], edge_index_b[0],
                           jnp.zeros((pad_e,), jnp.int32)])
    dst = jnp.concatenate([edge_index_a[1], edge_index_b[1],
                           jnp.full((pad_e,), DUMP, jnp.int32)])
    ts_p = jnp.pad(node_ts, (0, TS_PAD - N_NODES))
    zeros = jnp.zeros((ZROWS, HCH), jnp.float32)

    srcc, effc, cnts = _mask_kernel(ts_p, src, dst)

    xp = jnp.pad(x, ((0, ACC_R - N_NODES), (0, 0)))
    parts1 = _scatter_kernel(xp, srcc, effc, cnts, zeros)
    h1 = _layer_tc(parts1, W1, b1)
    parts2 = _scatter_kernel(h1, srcc, effc, cnts, zeros)
    out = _head_tc(parts2, W2, b2, fc1_W, fc1_b, fc2_W, fc2_b)
    return out[:N_NODES]
